# baseline (speedup 1.0000x reference)
import functools

import jax
import jax.numpy as jnp
from jax import lax
from jax.experimental import pallas as pl
from jax.experimental.pallas import tpu as pltpu
from jax.experimental.pallas import tpu_sc as plsc


def _topk_body(adj_ref, xf_ref, idx_ref, y_ref, *, rows_per_batch, blocks_per_batch, k, c):
    blk = pl.program_id(0)

    @pl.when(blk == 0)
    def _():
        _y_compute(xf_ref, y_ref, c)

    v = adj_ref[...]
    r, n = v.shape
    col = lax.broadcasted_iota(jnp.int32, (r, n), 1)
    slot = lax.broadcasted_iota(jnp.int32, (r, k), 1)
    base = (blk // blocks_per_batch) * rows_per_batch

    def body(t, carry):
        vv, acc = carry
        i = jnp.argmax(vv, axis=1, keepdims=True)
        acc = acc + jnp.where(slot == t, i + base, 0)
        vv = jnp.where(col == i, -1.0, vv)
        return vv, acc

    _, acc = lax.fori_loop(
        0, k, body, (v, jnp.zeros((r, k), jnp.int32)), unroll=32
    )
    idx_ref[...] = acc


def _topk_and_y(adj, xf, k, rows_per_batch, c, block_rows=512):
    rows, n = adj.shape
    nrows = xf.shape[0]
    grid = rows // block_rows
    return pl.pallas_call(
        functools.partial(
            _topk_body,
            rows_per_batch=rows_per_batch,
            blocks_per_batch=rows_per_batch // block_rows,
            k=k,
            c=c,
        ),
        grid=(grid,),
        in_specs=[
            pl.BlockSpec((block_rows, n), lambda i: (i, 0)),
            pl.BlockSpec((nrows, 3 * c), lambda i: (0, 0)),
        ],
        out_specs=[
            pl.BlockSpec((block_rows, k), lambda i: (i, 0)),
            pl.BlockSpec((nrows, 128), lambda i: (0, 0)),
        ],
        out_shape=[
            jax.ShapeDtypeStruct((rows, k), jnp.int32),
            jax.ShapeDtypeStruct((nrows, 128), jnp.float32),
        ],
    )(adj, xf)


def _y_compute(x_ref, y_ref, c):
    xb = x_ref[...]
    x0 = xb[:, 0:c]
    xc = jnp.clip(x0, -1.0, 1.0)
    t = 2.0 * jnp.arctan2(jnp.sqrt(1.0 - xc), jnp.sqrt(1.0 + xc))
    s = t / (jnp.sin(t) + 1e-4)
    y0 = (x0 - jnp.cos(t)) * s
    y1 = xb[:, c : 2 * c] * s
    y2 = xb[:, 2 * c : 3 * c] * s
    pad = jnp.zeros((xb.shape[0], 128 - 3 * c), jnp.float32)
    y_ref[...] = jnp.concatenate([y0, y1, y2, pad], axis=1)


def _sc_gather_reduce(y, idx_flat, wrow, k):
    rows, w = y.shape
    wa = wrow.shape[1]
    info = plsc.get_sparse_core_info()
    nw = info.num_cores * info.num_subcores
    nodes_per_w = rows // nw
    chunk = 8
    n_chunks = nodes_per_w // chunk
    mesh = plsc.VectorSubcoreMesh(core_axis_name="c", subcore_axis_name="s")

    @functools.partial(
        pl.kernel,
        mesh=mesh,
        out_type=jax.ShapeDtypeStruct((rows, wa), jnp.float32),
        scratch_types=[
            pltpu.VMEM((nodes_per_w * k,), jnp.int32),
            pltpu.VMEM((2, chunk * k, w), jnp.float32),
            pltpu.VMEM((k, wa), jnp.float32),
            pltpu.VMEM((chunk, wa), jnp.float32),
            pltpu.SemaphoreType.DMA,
            pltpu.SemaphoreType.DMA,
        ],
    )
    def body(
        y_hbm, idx_hbm, wrow_hbm, out_hbm, idx_v, rows_v, wrow_v, acc_v, sem0, sem1
    ):
        wid = lax.axis_index("s") * info.num_cores + lax.axis_index("c")
        pltpu.sync_copy(wrow_hbm, wrow_v)
        pltpu.sync_copy(idx_hbm.at[pl.ds(wid * nodes_per_w * k, nodes_per_w * k)], idx_v)

        def fire(ch, buf, sem):
            pltpu.async_copy(
                y_hbm.at[idx_v.at[pl.ds(ch * chunk * k, chunk * k)]],
                rows_v.at[buf],
                sem,
            )

        def drain(buf, sem):
            pltpu.make_async_copy(
                y_hbm.at[idx_v.at[pl.ds(0, chunk * k)]], rows_v.at[buf], sem
            ).wait()

        def compute(ch, buf):
            def pair_comp(i2, _):
                ia = i2 * 2
                ib = ia + 1
                z = jnp.zeros((16,), jnp.float32)
                a0e, a0o, a1e, a1o = z, z, z, z
                b0e, b0o, b1e, b1o = z, z, z, z
                for j in range(k):
                    w0 = wrow_v[j, pl.ds(0, 16)]
                    w1 = wrow_v[j, pl.ds(16, 16)]
                    ra0 = rows_v[buf, ia * k + j, pl.ds(0, 16)]
                    ra1 = rows_v[buf, ia * k + j, pl.ds(16, 16)]
                    rb0 = rows_v[buf, ib * k + j, pl.ds(0, 16)]
                    rb1 = rows_v[buf, ib * k + j, pl.ds(16, 16)]
                    if j % 2 == 0:
                        a0e = a0e + ra0 * w0
                        a1e = a1e + ra1 * w1
                        b0e = b0e + rb0 * w0
                        b1e = b1e + rb1 * w1
                    else:
                        a0o = a0o + ra0 * w0
                        a1o = a1o + ra1 * w1
                        b0o = b0o + rb0 * w0
                        b1o = b1o + rb1 * w1
                acc_v[ia, pl.ds(0, 16)] = a0e + a0o
                acc_v[ia, pl.ds(16, 16)] = a1e + a1o
                acc_v[ib, pl.ds(0, 16)] = b0e + b0o
                acc_v[ib, pl.ds(16, 16)] = b1e + b1o
                return 0

            lax.fori_loop(0, chunk // 2, pair_comp, 0)
            node0 = wid * nodes_per_w + ch * chunk
            pltpu.sync_copy(acc_v, out_hbm.at[pl.ds(node0, chunk)])

        fire(0, 0, sem0)

        def pair_body(p, _):
            ch0 = p * 2
            drain(0, sem0)
            fire(ch0 + 1, 1, sem1)
            compute(ch0, 0)
            drain(1, sem1)

            @pl.when(p < n_chunks // 2 - 1)
            def _():
                fire(ch0 + 2, 0, sem0)

            compute(ch0 + 1, 1)
            return 0

        lax.fori_loop(0, n_chunks // 2, pair_body, 0)

    return body(y, idx_flat, wrow)


def _fin_body(w_ref, w2_ref, o_ref, *, m):
    wv = w_ref[...]
    w2 = w2_ref[...]
    ws = jnp.dot(wv, w2, preferred_element_type=jnp.float32)
    a = ws[:, 0:m]
    b = ws[:, m : 2 * m]
    c = ws[:, 2 * m : 3 * m]
    vmag = jnp.sqrt(a * a + b * b + c * c)
    sv = jnp.sin(vmag) / jnp.maximum(vmag, 1e-12)
    o_ref[...] = jnp.concatenate(
        [jnp.cos(vmag) + sv * a, sv * b, sv * c], axis=1
    )


def _finale(weighted, w2blk, m):
    rows = weighted.shape[0]
    return pl.pallas_call(
        functools.partial(_fin_body, m=m),
        out_shape=jax.ShapeDtypeStruct((rows, 3 * m), jnp.float32),
    )(weighted, w2blk)


def kernel(x, adj_mtr, w1, w2):
    B, N, D, C = x.shape
    k = w1.shape[1]
    m = w2.shape[0]

    adj = adj_mtr.reshape(B * N, N)
    xf = x.reshape(B * N, D * C)
    idx, y = _topk_and_y(adj, xf, k, rows_per_batch=N, c=C)

    w1n = w1 * w1
    w1n = w1n / jnp.sum(w1n, axis=1, keepdims=True)
    wrow = jnp.concatenate(
        [jnp.tile(w1n.T, (1, D)), jnp.zeros((k, C), jnp.float32)], axis=1
    ) / float(k)

    weighted = _sc_gather_reduce(y, idx.reshape(-1), wrow, k)

    w2n = w2 * w2
    w2n = (w2n / jnp.sum(w2n, axis=1, keepdims=True)).T
    w2blk = jnp.zeros((4 * C, D * m), jnp.float32)
    for d in range(D):
        w2blk = w2blk.at[d * C : (d + 1) * C, d * m : (d + 1) * m].set(w2n)

    out = _finale(weighted, w2blk, m)
    return out.reshape(B, N, D, m)


# speedup vs baseline: 1.6872x; 1.6872x over previous
import functools

import jax
import jax.numpy as jnp
from jax import lax
from jax.experimental import pallas as pl
from jax.experimental.pallas import tpu as pltpu
from jax.experimental.pallas import tpu_sc as plsc


def _topk_body(adj_ref, idx_ref, *, rows_per_batch, blocks_per_batch, k):
    blk = pl.program_id(0)
    v = adj_ref[...]
    r, n = v.shape
    col = lax.broadcasted_iota(jnp.int32, (r, n), 1)
    slot = lax.broadcasted_iota(jnp.int32, (r, k), 1)
    base = (blk // blocks_per_batch) * rows_per_batch

    def body(t, carry):
        vv, acc = carry
        i = jnp.argmax(vv, axis=1, keepdims=True)
        acc = acc + jnp.where(slot == t, i + base, 0)
        vv = jnp.where(col == i, -1.0, vv)
        return vv, acc

    _, acc = lax.fori_loop(
        0, k, body, (v, jnp.zeros((r, k), jnp.int32)), unroll=32
    )
    idx_ref[...] = acc


def _topk(adj, k, rows_per_batch, block_rows=512):
    rows, n = adj.shape
    grid = rows // block_rows
    return pl.pallas_call(
        functools.partial(
            _topk_body,
            rows_per_batch=rows_per_batch,
            blocks_per_batch=rows_per_batch // block_rows,
            k=k,
        ),
        grid=(grid,),
        in_specs=[pl.BlockSpec((block_rows, n), lambda i: (i, 0))],
        out_specs=pl.BlockSpec((block_rows, k), lambda i: (i, 0)),
        out_shape=jax.ShapeDtypeStruct((rows, k), jnp.int32),
    )(adj)


def _y_body(x_ref, y_ref, *, c):
    xb = x_ref[...]
    x0 = xb[:, 0:c]
    xc = jnp.clip(x0, -1.0, 1.0)
    t = 2.0 * jnp.arctan2(jnp.sqrt(1.0 - xc), jnp.sqrt(1.0 + xc))
    s = t / (jnp.sin(t) + 1e-4)
    y0 = (x0 - jnp.cos(t)) * s
    y1 = xb[:, c : 2 * c] * s
    y2 = xb[:, 2 * c : 3 * c] * s
    pad = jnp.zeros((xb.shape[0], 128 - 3 * c), jnp.float32)
    y_ref[...] = jnp.concatenate([y0, y1, y2, pad], axis=1)


def _point_transform(xf, c):
    rows = xf.shape[0]
    return pl.pallas_call(
        functools.partial(_y_body, c=c),
        out_shape=jax.ShapeDtypeStruct((rows, 128), jnp.float32),
    )(xf)


def _sc_gather_reduce(y, idx_flat, wrow, k):
    rows, w = y.shape
    wa = wrow.shape[1]
    info = plsc.get_sparse_core_info()
    nw = info.num_cores * info.num_subcores
    nodes_per_w = rows // nw
    chunk = 8
    n_chunks = nodes_per_w // chunk
    mesh = plsc.VectorSubcoreMesh(core_axis_name="c", subcore_axis_name="s")

    @functools.partial(
        pl.kernel,
        mesh=mesh,
        out_type=jax.ShapeDtypeStruct((rows, wa), jnp.float32),
        scratch_types=[
            pltpu.VMEM((nodes_per_w * k,), jnp.int32),
            pltpu.VMEM((2, chunk * k, w), jnp.float32),
            pltpu.VMEM((k, wa), jnp.float32),
            pltpu.VMEM((chunk, wa), jnp.float32),
            pltpu.SemaphoreType.DMA,
            pltpu.SemaphoreType.DMA,
        ],
    )
    def body(
        y_hbm, idx_hbm, wrow_hbm, out_hbm, idx_v, rows_v, wrow_v, acc_v, sem0, sem1
    ):
        wid = lax.axis_index("s") * info.num_cores + lax.axis_index("c")
        pltpu.sync_copy(wrow_hbm, wrow_v)
        pltpu.sync_copy(idx_hbm.at[pl.ds(wid * nodes_per_w * k, nodes_per_w * k)], idx_v)

        def fire(ch, buf, sem):
            pltpu.async_copy(
                y_hbm.at[idx_v.at[pl.ds(ch * chunk * k, chunk * k)]],
                rows_v.at[buf],
                sem,
            )

        def drain(buf, sem):
            pltpu.make_async_copy(
                y_hbm.at[idx_v.at[pl.ds(0, chunk * k)]], rows_v.at[buf], sem
            ).wait()

        def compute(ch, buf):
            def pair_comp(i2, _):
                ia = i2 * 2
                ib = ia + 1
                z = jnp.zeros((16,), jnp.float32)
                a0e, a0o, a1e, a1o = z, z, z, z
                b0e, b0o, b1e, b1o = z, z, z, z
                for j in range(k):
                    w0 = wrow_v[j, pl.ds(0, 16)]
                    w1 = wrow_v[j, pl.ds(16, 16)]
                    ra0 = rows_v[buf, ia * k + j, pl.ds(0, 16)]
                    ra1 = rows_v[buf, ia * k + j, pl.ds(16, 16)]
                    rb0 = rows_v[buf, ib * k + j, pl.ds(0, 16)]
                    rb1 = rows_v[buf, ib * k + j, pl.ds(16, 16)]
                    if j % 2 == 0:
                        a0e = a0e + ra0 * w0
                        a1e = a1e + ra1 * w1
                        b0e = b0e + rb0 * w0
                        b1e = b1e + rb1 * w1
                    else:
                        a0o = a0o + ra0 * w0
                        a1o = a1o + ra1 * w1
                        b0o = b0o + rb0 * w0
                        b1o = b1o + rb1 * w1
                acc_v[ia, pl.ds(0, 16)] = a0e + a0o
                acc_v[ia, pl.ds(16, 16)] = a1e + a1o
                acc_v[ib, pl.ds(0, 16)] = b0e + b0o
                acc_v[ib, pl.ds(16, 16)] = b1e + b1o
                return 0

            lax.fori_loop(0, chunk // 2, pair_comp, 0)
            node0 = wid * nodes_per_w + ch * chunk
            pltpu.sync_copy(acc_v, out_hbm.at[pl.ds(node0, chunk)])

        fire(0, 0, sem0)

        def pair_body(p, _):
            ch0 = p * 2
            drain(0, sem0)
            fire(ch0 + 1, 1, sem1)
            compute(ch0, 0)
            drain(1, sem1)

            @pl.when(p < n_chunks // 2 - 1)
            def _():
                fire(ch0 + 2, 0, sem0)

            compute(ch0 + 1, 1)
            return 0

        lax.fori_loop(0, n_chunks // 2, pair_body, 0)

    return body(y, idx_flat, wrow)


def _fin_body(w_ref, w2_ref, o_ref, *, m):
    wv = w_ref[...]
    w2 = w2_ref[...]
    ws = jnp.dot(wv, w2, preferred_element_type=jnp.float32)
    a = ws[:, 0:m]
    b = ws[:, m : 2 * m]
    c = ws[:, 2 * m : 3 * m]
    vmag = jnp.sqrt(a * a + b * b + c * c)
    sv = jnp.sin(vmag) / jnp.maximum(vmag, 1e-12)
    o_ref[...] = jnp.concatenate(
        [jnp.cos(vmag) + sv * a, sv * b, sv * c], axis=1
    )


def _finale(weighted, w2blk, m):
    rows = weighted.shape[0]
    return pl.pallas_call(
        functools.partial(_fin_body, m=m),
        out_shape=jax.ShapeDtypeStruct((rows, 3 * m), jnp.float32),
    )(weighted, w2blk)


def kernel(x, adj_mtr, w1, w2):
    B, N, D, C = x.shape
    k = w1.shape[1]
    m = w2.shape[0]

    adj = adj_mtr.reshape(B * N, N)
    idx = _topk(adj, k, rows_per_batch=N)

    xf = x.reshape(B * N, D * C)
    y = _point_transform(xf, C)

    w1n = w1 * w1
    w1n = w1n / jnp.sum(w1n, axis=1, keepdims=True)
    wrow = jnp.concatenate(
        [jnp.tile(w1n.T, (1, D)), jnp.zeros((k, C), jnp.float32)], axis=1
    ) / float(k)

    weighted = _sc_gather_reduce(y, idx.reshape(-1), wrow, k)

    w2n = w2 * w2
    w2n = (w2n / jnp.sum(w2n, axis=1, keepdims=True)).T
    w2blk = jnp.zeros((4 * C, D * m), jnp.float32)
    for d in range(D):
        w2blk = w2blk.at[d * C : (d + 1) * C, d * m : (d + 1) * m].set(w2n)

    out = _finale(weighted, w2blk, m)
    return out.reshape(B, N, D, m)
